# baseline (speedup 1.0000x reference)
import functools

import jax
import jax.numpy as jnp
from jax import lax
from jax.experimental import pallas as pl
from jax.experimental.pallas import tpu as pltpu
from jax.experimental.pallas import tpu_sc as plsc

N = 50000
E = 800000
F_IN = 16
H = 128
B = 256
T = 5

NC = 2
NS = 16
L = 16
NW = NC * NS

W_T = 784
ACC_T = W_T + 8
DUMP = W_T
ROUNDS = 2
NPAD = ROUNDS * NW * W_T
G = 64
CH = 1536
NCHF = E // CH
TAIL = E - NCHF * CH
MB = CH + 2 * G

def _worker_id():
  return lax.axis_index("s") * NC + lax.axis_index("c")


EPT = E // NW
_EPT_FULL = (EPT // L) * L
_EPT_REM = EPT - _EPT_FULL


def _deg_body(dst_hbm, part_hbm, ebuf, hist):
  w = _worker_id()
  zv = jnp.zeros((L,), jnp.float32)

  def zero_body(i, _):
    hist[pl.ds(i * L, L)] = zv
    return 0

  lax.fori_loop(0, N // L, zero_body, 0, unroll=4)

  pltpu.sync_copy(dst_hbm.at[pl.ds(w * EPT, EPT)], ebuf.at[pl.ds(0, EPT)])
  ones = jnp.ones((L,), jnp.float32)

  def edge_body(i, _):
    idx = ebuf[pl.ds(i * L, L)]
    plsc.addupdate_scatter(hist, [idx], ones)
    return 0

  lax.fori_loop(0, _EPT_FULL // L, edge_body, 0, unroll=4)
  lanes = lax.iota(jnp.int32, L)
  m = lanes < _EPT_REM
  idx = jnp.where(m, ebuf[pl.ds(_EPT_FULL, L)], 0)
  plsc.addupdate_scatter(hist, [idx], ones, mask=m)

  pltpu.sync_copy(hist, part_hbm.at[w])


def _scatter_body(hw_hbm, dst_hbm, src_hbm, agg_hbm,
                  dbuf, sbuf, mdst, msrc, idx2, rowbuf, acc,
                  gsem0, gsem1, csem0, csem1):
  w = _worker_id()
  zv = jnp.zeros((L,), jnp.float32)
  dump_i = jnp.full((L,), DUMP, jnp.int32)
  zero_i = jnp.zeros((L,), jnp.int32)
  csems = (csem0, csem1)

  def issue_chunk(ci, b):
    off = ci * CH
    pltpu.async_copy(dst_hbm.at[pl.ds(off, CH)], dbuf.at[b], csems[b])
    pltpu.async_copy(src_hbm.at[pl.ds(off, CH)], sbuf.at[b], csems[b])

  def wait_chunk(ci, b):
    off = ci * CH
    pltpu.make_async_copy(dst_hbm.at[pl.ds(off, CH)], dbuf.at[b], csems[b]).wait()
    pltpu.make_async_copy(src_hbm.at[pl.ds(off, CH)], sbuf.at[b], csems[b]).wait()

  def accumulate(b, j):
    def acc_k8(k8, _):
      dl = mdst[pl.ds(j * G + k8 * L, L)]
      for k in range(L):
        d = dl[k]
        row = k8 * L + k
        for q in range(H // L):
          acc[d, pl.ds(q * L, L)] = (
              acc[d, pl.ds(q * L, L)] + rowbuf[b, row, pl.ds(q * L, L)])
      return 0

    lax.fori_loop(0, G // L, acc_k8, 0)

  def stage_idx(j, b):
    base = j * G
    for k in range(G // L):
      idx2[b, pl.ds(k * L, L)] = msrc[pl.ds(base + k * L, L)]

  def make_flush_pairs(ng):
    def flush_pair(p, _):
      j0 = 2 * p
      j1 = j0 + 1
      stage_idx(j0, 0)
      g0 = pltpu.async_copy(hw_hbm.at[idx2.at[0]], rowbuf.at[0], gsem0)

      @pl.when(j1 < ng)
      def _():
        stage_idx(j1, 1)
        pltpu.async_copy(hw_hbm.at[idx2.at[1]], rowbuf.at[1], gsem1)

      g0.wait()
      accumulate(0, j0)

      @pl.when(j1 < ng)
      def _():
        pltpu.make_async_copy(hw_hbm.at[idx2.at[1]], rowbuf.at[1], gsem1).wait()
        accumulate(1, j1)

      return 0

    return flush_pair

  def process_data(b, ngroups, lo, cur):
    def filter_group(g, cur):
      d = dbuf[b, pl.ds(g * L, L)]
      sv = sbuf[b, pl.ds(g * L, L)]
      m = (d >= lo) & (d < lo + W_T)
      plsc.store_compressed(mdst.at[pl.ds(cur, L)], d - lo, mask=m)
      plsc.store_compressed(msrc.at[pl.ds(cur, L)], sv, mask=m)
      return cur + jnp.sum(m.astype(jnp.int32))

    cur = lax.fori_loop(0, ngroups, filter_group, cur)
    ngf = cur // G
    lax.fori_loop(0, (ngf + 1) // 2, make_flush_pairs(ngf), 0)

    @pl.when(ngf > 0)
    def _():
      base = ngf * G
      for k in range(G // L):
        td = mdst[pl.ds(base + k * L, L)]
        ts = msrc[pl.ds(base + k * L, L)]
        mdst[pl.ds(k * L, L)] = td
        msrc[pl.ds(k * L, L)] = ts

    return cur - ngf * G

  def zero_acc():
    def zrow(i, _):
      for k in range(H // L):
        acc[i, pl.ds(k * L, L)] = zv
      return 0

    lax.fori_loop(0, W_T, zrow, 0, unroll=2)

  for r in range(ROUNDS):
    lo = (r * NW + w) * W_T
    zero_acc()
    issue_chunk(0, 0)

    def chunk_pair(p, cur):
      c0 = 2 * p
      wait_chunk(c0, 0)
      issue_chunk(c0 + 1, 1)
      cur = process_data(0, CH // L, lo, cur)
      wait_chunk(c0 + 1, 1)

      @pl.when(c0 + 2 < NCHF)
      def _():
        issue_chunk(c0 + 2, 0)

      cur = process_data(1, CH // L, lo, cur)
      return cur

    cur = lax.fori_loop(0, NCHF // 2, chunk_pair, jnp.int32(0))
    pltpu.sync_copy(dst_hbm.at[pl.ds(NCHF * CH, TAIL)], dbuf.at[0, pl.ds(0, TAIL)])
    pltpu.sync_copy(src_hbm.at[pl.ds(NCHF * CH, TAIL)], sbuf.at[0, pl.ds(0, TAIL)])
    cur = process_data(0, TAIL // L, lo, cur)

    for k in range(G // L):
      mdst[pl.ds(cur + k * L, L)] = dump_i
      msrc[pl.ds(cur + k * L, L)] = zero_i
    ng_last = (cur + (G - 1)) // G
    lax.fori_loop(0, ng_last, make_flush_pairs(ng_last), 0)

    pltpu.sync_copy(acc.at[pl.ds(0, W_T)], agg_hbm.at[pl.ds(lo, W_T)])


NPT = 1552
PTAIL = N - NPT * NW
PROWS = B + 8
CHP = 128


def _pool_body(h_hbm, batch_hbm, psum_hbm, pmax_hbm, pcnt_hbm,
               hbuf, bbv, asum, amax, acnt):
  w = _worker_id()
  zv = jnp.zeros((L,), jnp.float32)
  ninf = jnp.full((L,), -3.0e38, jnp.float32)
  onev = jnp.ones((L,), jnp.float32)

  def init_body(i, _):
    for k in range(H // L):
      asum[i, pl.ds(k * L, L)] = zv
      amax[i, pl.ds(k * L, L)] = ninf
    acnt[i, pl.ds(0, L)] = zv
    return 0

  lax.fori_loop(0, PROWS, init_body, 0, unroll=2)

  def do_chunk(base, nrows):
    pltpu.sync_copy(h_hbm.at[pl.ds(base, nrows)], hbuf.at[pl.ds(0, nrows)])
    pltpu.sync_copy(batch_hbm.at[pl.ds(base, nrows)], bbv.at[pl.ds(0, nrows)])

    def group_body(g, _):
      bvec = bbv[pl.ds(g * L, L)]
      for k in range(L):
        b = bvec[k]
        row = g * L + k
        for q in range(H // L):
          v = hbuf[row, pl.ds(q * L, L)]
          asum[b, pl.ds(q * L, L)] = asum[b, pl.ds(q * L, L)] + v
          amax[b, pl.ds(q * L, L)] = jnp.maximum(amax[b, pl.ds(q * L, L)], v)
        acnt[b, pl.ds(0, L)] = acnt[b, pl.ds(0, L)] + onev
      return 0

    lax.fori_loop(0, nrows // L, group_body, 0)

  base0 = w * NPT

  def chunk_body(ci, _):
    do_chunk(base0 + ci * CHP, CHP)
    return 0

  lax.fori_loop(0, NPT // CHP, chunk_body, 0)
  do_chunk(base0 + (NPT // CHP) * CHP, NPT - (NPT // CHP) * CHP)

  @pl.when(w == NW - 1)
  def _tail():
    tb = NW * NPT

    def tail_chunk(ci, _):
      do_chunk(tb + ci * CHP, CHP)
      return 0

    lax.fori_loop(0, PTAIL // CHP, tail_chunk, 0)
    do_chunk(tb + (PTAIL // CHP) * CHP, PTAIL - (PTAIL // CHP) * CHP)

  pltpu.sync_copy(asum, psum_hbm.at[w])
  pltpu.sync_copy(amax, pmax_hbm.at[w])
  pltpu.sync_copy(acnt, pcnt_hbm.at[w])


@functools.cache
def _sc_kernels():
  mesh = plsc.VectorSubcoreMesh(
      core_axis_name="c", subcore_axis_name="s",
      num_cores=NC, num_subcores=NS)
  cp = pltpu.CompilerParams(
      needs_layout_passes=False, use_tc_tiling_on_sc=False)
  deg = pl.kernel(
      _deg_body,
      out_type=jax.ShapeDtypeStruct((NW, N), jnp.float32),
      mesh=mesh,
      scratch_types=[
          pltpu.VMEM((_EPT_FULL + L,), jnp.int32),
          pltpu.VMEM((N,), jnp.float32),
      ],
      compiler_params=cp,
  )
  scatter = pl.kernel(
      _scatter_body,
      out_type=jax.ShapeDtypeStruct((NPAD, H), jnp.float32),
      mesh=mesh,
      scratch_types=[
          pltpu.VMEM((2, CH), jnp.int32),
          pltpu.VMEM((2, CH), jnp.int32),
          pltpu.VMEM((MB,), jnp.int32),
          pltpu.VMEM((MB,), jnp.int32),
          pltpu.VMEM((2, G), jnp.int32),
          pltpu.VMEM((2, G, H), jnp.float32),
          pltpu.VMEM((ACC_T, H), jnp.float32),
          pltpu.SemaphoreType.DMA,
          pltpu.SemaphoreType.DMA,
          pltpu.SemaphoreType.DMA,
          pltpu.SemaphoreType.DMA,
      ],
      compiler_params=cp,
  )
  pool = pl.kernel(
      _pool_body,
      out_type=(
          jax.ShapeDtypeStruct((NW, PROWS, H), jnp.float32),
          jax.ShapeDtypeStruct((NW, PROWS, H), jnp.float32),
          jax.ShapeDtypeStruct((NW, PROWS, L), jnp.float32),
      ),
      mesh=mesh,
      scratch_types=[
          pltpu.VMEM((CHP, H), jnp.float32),
          pltpu.VMEM((CHP,), jnp.int32),
          pltpu.VMEM((PROWS, H), jnp.float32),
          pltpu.VMEM((PROWS, H), jnp.float32),
          pltpu.VMEM((PROWS, L), jnp.float32),
      ],
      compiler_params=cp,
  )
  return deg, scatter, pool


RB = 1000
GRID = N // RB


def _prep_body(x_ref, pt_ref, w0_ref, dinv_ref, hw_ref):
  deg = jnp.sum(pt_ref[...], axis=1, keepdims=True) + 1.0
  dinv = lax.rsqrt(jnp.maximum(deg, 1.0))
  dinv_ref[...] = dinv
  hw = jnp.dot(x_ref[...], w0_ref[...], preferred_element_type=jnp.float32)
  hw_ref[...] = hw * dinv


_prep = pl.pallas_call(
    _prep_body,
    grid=(GRID,),
    in_specs=[
        pl.BlockSpec((RB, F_IN), lambda i: (i, 0)),
        pl.BlockSpec((RB, NW), lambda i: (i, 0)),
        pl.BlockSpec((F_IN, H), lambda i: (0, 0)),
    ],
    out_specs=[
        pl.BlockSpec((RB, 1), lambda i: (i, 0)),
        pl.BlockSpec((RB, H), lambda i: (i, 0)),
    ],
    out_shape=[
        jax.ShapeDtypeStruct((N, 1), jnp.float32),
        jax.ShapeDtypeStruct((N, H), jnp.float32),
    ],
)


def _post_body(agg_ref, hwp_ref, dinv_ref, b_ref, z_ref, st_ref):
  i = pl.program_id(0)
  zv = dinv_ref[...] * (agg_ref[...] + hwp_ref[...]) + b_ref[...]
  z_ref[...] = zv

  @pl.when(i == 0)
  def _():
    st_ref[...] = jnp.zeros((8, H), jnp.float32)

  st_ref[0:1, :] += jnp.sum(zv, axis=0, keepdims=True)
  st_ref[1:2, :] += jnp.sum(zv * zv, axis=0, keepdims=True)


_post = pl.pallas_call(
    _post_body,
    grid=(GRID,),
    in_specs=[
        pl.BlockSpec((RB, H), lambda i: (i, 0)),
        pl.BlockSpec((RB, H), lambda i: (i, 0)),
        pl.BlockSpec((RB, 1), lambda i: (i, 0)),
        pl.BlockSpec((1, H), lambda i: (0, 0)),
    ],
    out_specs=[
        pl.BlockSpec((RB, H), lambda i: (i, 0)),
        pl.BlockSpec((8, H), lambda i: (0, 0)),
    ],
    out_shape=[
        jax.ShapeDtypeStruct((N, H), jnp.float32),
        jax.ShapeDtypeStruct((8, H), jnp.float32),
    ],
)


def _bn_common(z_ref, st_ref, g_ref, be_ref):
  mu = st_ref[0:1, :] * (1.0 / N)
  ms = st_ref[1:2, :] * (1.0 / N)
  var = ms - mu * mu
  scale = g_ref[...] * lax.rsqrt(var + 1e-5)
  return jnp.maximum((z_ref[...] - mu) * scale + be_ref[...], 0.0)


def _bn_body(z_ref, st_ref, g_ref, be_ref, w_ref, dinv_ref, hwn_ref):
  h = _bn_common(z_ref, st_ref, g_ref, be_ref)
  hwn_ref[...] = jnp.dot(h, w_ref[...], preferred_element_type=jnp.float32) * dinv_ref[...]


_bn = pl.pallas_call(
    _bn_body,
    grid=(GRID,),
    in_specs=[
        pl.BlockSpec((RB, H), lambda i: (i, 0)),
        pl.BlockSpec((8, H), lambda i: (0, 0)),
        pl.BlockSpec((1, H), lambda i: (0, 0)),
        pl.BlockSpec((1, H), lambda i: (0, 0)),
        pl.BlockSpec((H, H), lambda i: (0, 0)),
        pl.BlockSpec((RB, 1), lambda i: (i, 0)),
    ],
    out_specs=pl.BlockSpec((RB, H), lambda i: (i, 0)),
    out_shape=jax.ShapeDtypeStruct((N, H), jnp.float32),
)


def _bn_last_body(z_ref, st_ref, g_ref, be_ref, h_ref):
  h_ref[...] = _bn_common(z_ref, st_ref, g_ref, be_ref)


_bn_last = pl.pallas_call(
    _bn_last_body,
    grid=(GRID,),
    in_specs=[
        pl.BlockSpec((RB, H), lambda i: (i, 0)),
        pl.BlockSpec((8, H), lambda i: (0, 0)),
        pl.BlockSpec((1, H), lambda i: (0, 0)),
        pl.BlockSpec((1, H), lambda i: (0, 0)),
    ],
    out_specs=pl.BlockSpec((RB, H), lambda i: (i, 0)),
    out_shape=jax.ShapeDtypeStruct((N, H), jnp.float32),
)


def _head_body(ps_ref, pm_ref, pc_ref, w1_ref, b1_ref, w2_ref, b2_ref,
               wo_ref, bo_ref, out_ref, s_sum, s_max, s_cnt):
  i = pl.program_id(0)

  @pl.when(i == 0)
  def _():
    s_sum[...] = ps_ref[0]
    s_max[...] = pm_ref[0]
    s_cnt[...] = pc_ref[0]

  @pl.when(i > 0)
  def _():
    s_sum[...] += ps_ref[0]
    s_max[...] = jnp.maximum(s_max[...], pm_ref[0])
    s_cnt[...] += pc_ref[0]

  @pl.when(i == NW - 1)
  def _():
    cnt = s_cnt[:, 0:1]
    mean = s_sum[...] / jnp.maximum(cnt, 1.0)
    mx = jnp.where(cnt > 0.0, s_max[...], 0.0)
    z = jnp.concatenate([mean, mx], axis=1)
    z = jnp.maximum(jnp.dot(z, w1_ref[...], preferred_element_type=jnp.float32)
                    + b1_ref[...], 0.0)
    z = jnp.maximum(jnp.dot(z, w2_ref[...], preferred_element_type=jnp.float32)
                    + b2_ref[...], 0.0)
    out_ref[...] = (jnp.dot(z, wo_ref[...], preferred_element_type=jnp.float32)
                    + bo_ref[...])


_head = pl.pallas_call(
    _head_body,
    grid=(NW,),
    in_specs=[
        pl.BlockSpec((1, B, H), lambda i: (i, 0, 0)),
        pl.BlockSpec((1, B, H), lambda i: (i, 0, 0)),
        pl.BlockSpec((1, B, L), lambda i: (i, 0, 0)),
        pl.BlockSpec((2 * H, H), lambda i: (0, 0)),
        pl.BlockSpec((1, H), lambda i: (0, 0)),
        pl.BlockSpec((H, H // 2), lambda i: (0, 0)),
        pl.BlockSpec((1, H // 2), lambda i: (0, 0)),
        pl.BlockSpec((H // 2, T), lambda i: (0, 0)),
        pl.BlockSpec((1, T), lambda i: (0, 0)),
    ],
    out_specs=pl.BlockSpec((B, T), lambda i: (0, 0)),
    out_shape=jax.ShapeDtypeStruct((B, T), jnp.float32),
    scratch_shapes=[
        pltpu.VMEM((B, H), jnp.float32),
        pltpu.VMEM((B, H), jnp.float32),
        pltpu.VMEM((B, L), jnp.float32),
    ],
)


def kernel(x, edge_index, batch, W0, b0, g0, be0, W1, b1, g1, be1,
           W2, b2, g2, be2, fc1W, fc1b, fc2W, fc2b, foW, fob):
  src = edge_index[0]
  dst = edge_index[1]
  _deg_kernel, _scatter_kernel, _pool_kernel = _sc_kernels()

  parts = _deg_kernel(dst)
  dinv, hw = _prep(x, parts.T, W0)

  layer_params = ((b0, g0, be0, W1), (b1, g1, be1, W2), (b2, g2, be2, None))
  h3 = None
  for li, (b, g, be, Wn) in enumerate(layer_params):
    agg = _scatter_kernel(hw, dst, src)
    z, stats = _post(agg, hw, dinv, b.reshape(1, H))
    if Wn is not None:
      hw = _bn(z, stats, g.reshape(1, H), be.reshape(1, H), Wn, dinv)
    else:
      h3 = _bn_last(z, stats, g.reshape(1, H), be.reshape(1, H))

  psum, pmax, pcnt = _pool_kernel(h3, batch)
  return _head(psum, pmax, pcnt,
               fc1W, fc1b.reshape(1, H), fc2W, fc2b.reshape(1, H // 2),
               foW, fob.reshape(1, T))


# speedup vs baseline: 1.1052x; 1.1052x over previous
import functools

import jax
import jax.numpy as jnp
from jax import lax
from jax.experimental import pallas as pl
from jax.experimental.pallas import tpu as pltpu
from jax.experimental.pallas import tpu_sc as plsc

N = 50000
E = 800000
F_IN = 16
H = 128
B = 256
T = 5

NC = 2
NS = 16
L = 16
NW = NC * NS

W_T = 784
ACC_T = W_T + 8
DUMP = W_T
ROUNDS = 2
NPAD = ROUNDS * NW * W_T
G = 64
CH = 1792
NCHF = E // CH
TAIL = E - NCHF * CH
MB = CH + 2 * G

def _worker_id():
  return lax.axis_index("s") * NC + lax.axis_index("c")


EPT = E // NW
_EPT_FULL = (EPT // L) * L
_EPT_REM = EPT - _EPT_FULL


def _deg_body(dst_hbm, part_hbm, ebuf, hist):
  w = _worker_id()
  zv = jnp.zeros((L,), jnp.float32)

  def zero_body(i, _):
    hist[pl.ds(i * L, L)] = zv
    return 0

  lax.fori_loop(0, N // L, zero_body, 0, unroll=4)

  pltpu.sync_copy(dst_hbm.at[pl.ds(w * EPT, EPT)], ebuf.at[pl.ds(0, EPT)])
  ones = jnp.ones((L,), jnp.float32)

  def edge_body(i, _):
    idx = ebuf[pl.ds(i * L, L)]
    plsc.addupdate_scatter(hist, [idx], ones)
    return 0

  lax.fori_loop(0, _EPT_FULL // L, edge_body, 0, unroll=4)
  lanes = lax.iota(jnp.int32, L)
  m = lanes < _EPT_REM
  idx = jnp.where(m, ebuf[pl.ds(_EPT_FULL, L)], 0)
  plsc.addupdate_scatter(hist, [idx], ones, mask=m)

  pltpu.sync_copy(hist, part_hbm.at[w])


def _scatter_body(hw_hbm, dst_hbm, src_hbm, agg_hbm,
                  dbuf, sbuf, mdst, msrc, idx2, rowbuf, acc,
                  gsem0, gsem1, csem0, csem1):
  w = _worker_id()
  zv = jnp.zeros((L,), jnp.float32)
  dump_i = jnp.full((L,), DUMP, jnp.int32)
  zero_i = jnp.zeros((L,), jnp.int32)
  csems = (csem0, csem1)

  def issue_chunk(ci, b):
    off = ci * CH
    pltpu.async_copy(dst_hbm.at[pl.ds(off, CH)], dbuf.at[b], csems[b])
    pltpu.async_copy(src_hbm.at[pl.ds(off, CH)], sbuf.at[b], csems[b])

  def wait_chunk(ci, b):
    off = ci * CH
    pltpu.make_async_copy(dst_hbm.at[pl.ds(off, CH)], dbuf.at[b], csems[b]).wait()
    pltpu.make_async_copy(src_hbm.at[pl.ds(off, CH)], sbuf.at[b], csems[b]).wait()

  def accumulate(b, j):
    def acc_k8(k8, _):
      dl = mdst[pl.ds(j * G + k8 * L, L)]
      for k in range(L):
        d = dl[k]
        row = k8 * L + k
        for q in range(H // L):
          plsc.addupdate(acc.at[d, pl.ds(q * L, L)],
                         rowbuf[b, row, pl.ds(q * L, L)])
      return 0

    lax.fori_loop(0, G // L, acc_k8, 0)

  def stage_idx(j, b):
    base = j * G
    for k in range(G // L):
      idx2[b, pl.ds(k * L, L)] = msrc[pl.ds(base + k * L, L)]

  def make_flush_pairs(ng):
    def flush_pair(p, _):
      j0 = 2 * p
      j1 = j0 + 1
      stage_idx(j0, 0)
      g0 = pltpu.async_copy(hw_hbm.at[idx2.at[0]], rowbuf.at[0], gsem0)

      @pl.when(j1 < ng)
      def _():
        stage_idx(j1, 1)
        pltpu.async_copy(hw_hbm.at[idx2.at[1]], rowbuf.at[1], gsem1)

      g0.wait()
      accumulate(0, j0)

      @pl.when(j1 < ng)
      def _():
        pltpu.make_async_copy(hw_hbm.at[idx2.at[1]], rowbuf.at[1], gsem1).wait()
        accumulate(1, j1)

      return 0

    return flush_pair

  def process_data(b, ngroups, lo, cur):
    def filter_group(g, cur):
      d = dbuf[b, pl.ds(g * L, L)]
      sv = sbuf[b, pl.ds(g * L, L)]
      dl = d - lo
      m = plsc.bitcast(dl, jnp.uint32) < jnp.uint32(W_T)
      plsc.store_compressed(mdst.at[pl.ds(cur, L)], dl, mask=m)
      plsc.store_compressed(msrc.at[pl.ds(cur, L)], sv, mask=m)
      return cur + jnp.sum(m.astype(jnp.int32))

    cur = lax.fori_loop(0, ngroups, filter_group, cur)
    ngf = cur // G
    lax.fori_loop(0, (ngf + 1) // 2, make_flush_pairs(ngf), 0)

    @pl.when(ngf > 0)
    def _():
      base = ngf * G
      for k in range(G // L):
        td = mdst[pl.ds(base + k * L, L)]
        ts = msrc[pl.ds(base + k * L, L)]
        mdst[pl.ds(k * L, L)] = td
        msrc[pl.ds(k * L, L)] = ts

    return cur - ngf * G

  def zero_acc():
    def zrow(i, _):
      for k in range(H // L):
        acc[i, pl.ds(k * L, L)] = zv
      return 0

    lax.fori_loop(0, W_T, zrow, 0, unroll=2)

  for r in range(ROUNDS):
    lo = (r * NW + w) * W_T
    zero_acc()
    issue_chunk(0, 0)

    def chunk_pair(p, cur):
      c0 = 2 * p
      wait_chunk(c0, 0)
      issue_chunk(c0 + 1, 1)
      cur = process_data(0, CH // L, lo, cur)
      wait_chunk(c0 + 1, 1)

      @pl.when(c0 + 2 < NCHF)
      def _():
        issue_chunk(c0 + 2, 0)

      cur = process_data(1, CH // L, lo, cur)
      return cur

    cur = lax.fori_loop(0, NCHF // 2, chunk_pair, jnp.int32(0))
    pltpu.sync_copy(dst_hbm.at[pl.ds(NCHF * CH, TAIL)], dbuf.at[0, pl.ds(0, TAIL)])
    pltpu.sync_copy(src_hbm.at[pl.ds(NCHF * CH, TAIL)], sbuf.at[0, pl.ds(0, TAIL)])
    cur = process_data(0, TAIL // L, lo, cur)

    for k in range(G // L):
      mdst[pl.ds(cur + k * L, L)] = dump_i
      msrc[pl.ds(cur + k * L, L)] = zero_i
    ng_last = (cur + (G - 1)) // G
    lax.fori_loop(0, ng_last, make_flush_pairs(ng_last), 0)

    pltpu.sync_copy(acc.at[pl.ds(0, W_T)], agg_hbm.at[pl.ds(lo, W_T)])


NPT = 1552
PTAIL = N - NPT * NW
PROWS = B + 8
CHP = 128


def _pool_body(h_hbm, batch_hbm, psum_hbm, pmax_hbm, pcnt_hbm,
               hbuf, bbv, asum, amax, acnt):
  w = _worker_id()
  zv = jnp.zeros((L,), jnp.float32)
  ninf = jnp.full((L,), -3.0e38, jnp.float32)
  onev = jnp.ones((L,), jnp.float32)

  def init_body(i, _):
    for k in range(H // L):
      asum[i, pl.ds(k * L, L)] = zv
      amax[i, pl.ds(k * L, L)] = ninf
    acnt[i, pl.ds(0, L)] = zv
    return 0

  lax.fori_loop(0, PROWS, init_body, 0, unroll=2)

  def do_chunk(base, nrows):
    pltpu.sync_copy(h_hbm.at[pl.ds(base, nrows)], hbuf.at[pl.ds(0, nrows)])
    pltpu.sync_copy(batch_hbm.at[pl.ds(base, nrows)], bbv.at[pl.ds(0, nrows)])

    def group_body(g, _):
      bvec = bbv[pl.ds(g * L, L)]
      for k in range(L):
        b = bvec[k]
        row = g * L + k
        for q in range(H // L):
          v = hbuf[row, pl.ds(q * L, L)]
          asum[b, pl.ds(q * L, L)] = asum[b, pl.ds(q * L, L)] + v
          amax[b, pl.ds(q * L, L)] = jnp.maximum(amax[b, pl.ds(q * L, L)], v)
        acnt[b, pl.ds(0, L)] = acnt[b, pl.ds(0, L)] + onev
      return 0

    lax.fori_loop(0, nrows // L, group_body, 0)

  base0 = w * NPT

  def chunk_body(ci, _):
    do_chunk(base0 + ci * CHP, CHP)
    return 0

  lax.fori_loop(0, NPT // CHP, chunk_body, 0)
  do_chunk(base0 + (NPT // CHP) * CHP, NPT - (NPT // CHP) * CHP)

  @pl.when(w == NW - 1)
  def _tail():
    tb = NW * NPT

    def tail_chunk(ci, _):
      do_chunk(tb + ci * CHP, CHP)
      return 0

    lax.fori_loop(0, PTAIL // CHP, tail_chunk, 0)
    do_chunk(tb + (PTAIL // CHP) * CHP, PTAIL - (PTAIL // CHP) * CHP)

  pltpu.sync_copy(asum, psum_hbm.at[w])
  pltpu.sync_copy(amax, pmax_hbm.at[w])
  pltpu.sync_copy(acnt, pcnt_hbm.at[w])


@functools.cache
def _sc_kernels():
  mesh = plsc.VectorSubcoreMesh(
      core_axis_name="c", subcore_axis_name="s",
      num_cores=NC, num_subcores=NS)
  cp = pltpu.CompilerParams(
      needs_layout_passes=False, use_tc_tiling_on_sc=False)
  deg = pl.kernel(
      _deg_body,
      out_type=jax.ShapeDtypeStruct((NW, N), jnp.float32),
      mesh=mesh,
      scratch_types=[
          pltpu.VMEM((_EPT_FULL + L,), jnp.int32),
          pltpu.VMEM((N,), jnp.float32),
      ],
      compiler_params=cp,
  )
  scatter = pl.kernel(
      _scatter_body,
      out_type=jax.ShapeDtypeStruct((NPAD, H), jnp.float32),
      mesh=mesh,
      scratch_types=[
          pltpu.VMEM((2, CH), jnp.int32),
          pltpu.VMEM((2, CH), jnp.int32),
          pltpu.VMEM((MB,), jnp.int32),
          pltpu.VMEM((MB,), jnp.int32),
          pltpu.VMEM((2, G), jnp.int32),
          pltpu.VMEM((2, G, H), jnp.float32),
          pltpu.VMEM((ACC_T, H), jnp.float32),
          pltpu.SemaphoreType.DMA,
          pltpu.SemaphoreType.DMA,
          pltpu.SemaphoreType.DMA,
          pltpu.SemaphoreType.DMA,
      ],
      compiler_params=cp,
  )
  pool = pl.kernel(
      _pool_body,
      out_type=(
          jax.ShapeDtypeStruct((NW, PROWS, H), jnp.float32),
          jax.ShapeDtypeStruct((NW, PROWS, H), jnp.float32),
          jax.ShapeDtypeStruct((NW, PROWS, L), jnp.float32),
      ),
      mesh=mesh,
      scratch_types=[
          pltpu.VMEM((CHP, H), jnp.float32),
          pltpu.VMEM((CHP,), jnp.int32),
          pltpu.VMEM((PROWS, H), jnp.float32),
          pltpu.VMEM((PROWS, H), jnp.float32),
          pltpu.VMEM((PROWS, L), jnp.float32),
      ],
      compiler_params=cp,
  )
  return deg, scatter, pool


RB = 1000
GRID = N // RB


def _prep_body(x_ref, pt_ref, w0_ref, dinv_ref, hw_ref):
  deg = jnp.sum(pt_ref[...], axis=1, keepdims=True) + 1.0
  dinv = lax.rsqrt(jnp.maximum(deg, 1.0))
  dinv_ref[...] = dinv
  hw = jnp.dot(x_ref[...], w0_ref[...], preferred_element_type=jnp.float32)
  hw_ref[...] = hw * dinv


_prep = pl.pallas_call(
    _prep_body,
    grid=(GRID,),
    in_specs=[
        pl.BlockSpec((RB, F_IN), lambda i: (i, 0)),
        pl.BlockSpec((RB, NW), lambda i: (i, 0)),
        pl.BlockSpec((F_IN, H), lambda i: (0, 0)),
    ],
    out_specs=[
        pl.BlockSpec((RB, 1), lambda i: (i, 0)),
        pl.BlockSpec((RB, H), lambda i: (i, 0)),
    ],
    out_shape=[
        jax.ShapeDtypeStruct((N, 1), jnp.float32),
        jax.ShapeDtypeStruct((N, H), jnp.float32),
    ],
)


def _post_body(agg_ref, hwp_ref, dinv_ref, b_ref, z_ref, st_ref):
  i = pl.program_id(0)
  zv = dinv_ref[...] * (agg_ref[...] + hwp_ref[...]) + b_ref[...]
  z_ref[...] = zv

  @pl.when(i == 0)
  def _():
    st_ref[...] = jnp.zeros((8, H), jnp.float32)

  st_ref[0:1, :] += jnp.sum(zv, axis=0, keepdims=True)
  st_ref[1:2, :] += jnp.sum(zv * zv, axis=0, keepdims=True)


_post = pl.pallas_call(
    _post_body,
    grid=(GRID,),
    in_specs=[
        pl.BlockSpec((RB, H), lambda i: (i, 0)),
        pl.BlockSpec((RB, H), lambda i: (i, 0)),
        pl.BlockSpec((RB, 1), lambda i: (i, 0)),
        pl.BlockSpec((1, H), lambda i: (0, 0)),
    ],
    out_specs=[
        pl.BlockSpec((RB, H), lambda i: (i, 0)),
        pl.BlockSpec((8, H), lambda i: (0, 0)),
    ],
    out_shape=[
        jax.ShapeDtypeStruct((N, H), jnp.float32),
        jax.ShapeDtypeStruct((8, H), jnp.float32),
    ],
)


def _bn_common(z_ref, st_ref, g_ref, be_ref):
  mu = st_ref[0:1, :] * (1.0 / N)
  ms = st_ref[1:2, :] * (1.0 / N)
  var = ms - mu * mu
  scale = g_ref[...] * lax.rsqrt(var + 1e-5)
  return jnp.maximum((z_ref[...] - mu) * scale + be_ref[...], 0.0)


def _bn_body(z_ref, st_ref, g_ref, be_ref, w_ref, dinv_ref, hwn_ref):
  h = _bn_common(z_ref, st_ref, g_ref, be_ref)
  hwn_ref[...] = jnp.dot(h, w_ref[...], preferred_element_type=jnp.float32) * dinv_ref[...]


_bn = pl.pallas_call(
    _bn_body,
    grid=(GRID,),
    in_specs=[
        pl.BlockSpec((RB, H), lambda i: (i, 0)),
        pl.BlockSpec((8, H), lambda i: (0, 0)),
        pl.BlockSpec((1, H), lambda i: (0, 0)),
        pl.BlockSpec((1, H), lambda i: (0, 0)),
        pl.BlockSpec((H, H), lambda i: (0, 0)),
        pl.BlockSpec((RB, 1), lambda i: (i, 0)),
    ],
    out_specs=pl.BlockSpec((RB, H), lambda i: (i, 0)),
    out_shape=jax.ShapeDtypeStruct((N, H), jnp.float32),
)


def _bn_last_body(z_ref, st_ref, g_ref, be_ref, h_ref):
  h_ref[...] = _bn_common(z_ref, st_ref, g_ref, be_ref)


_bn_last = pl.pallas_call(
    _bn_last_body,
    grid=(GRID,),
    in_specs=[
        pl.BlockSpec((RB, H), lambda i: (i, 0)),
        pl.BlockSpec((8, H), lambda i: (0, 0)),
        pl.BlockSpec((1, H), lambda i: (0, 0)),
        pl.BlockSpec((1, H), lambda i: (0, 0)),
    ],
    out_specs=pl.BlockSpec((RB, H), lambda i: (i, 0)),
    out_shape=jax.ShapeDtypeStruct((N, H), jnp.float32),
)


def _head_body(ps_ref, pm_ref, pc_ref, w1_ref, b1_ref, w2_ref, b2_ref,
               wo_ref, bo_ref, out_ref, s_sum, s_max, s_cnt):
  i = pl.program_id(0)

  @pl.when(i == 0)
  def _():
    s_sum[...] = ps_ref[0]
    s_max[...] = pm_ref[0]
    s_cnt[...] = pc_ref[0]

  @pl.when(i > 0)
  def _():
    s_sum[...] += ps_ref[0]
    s_max[...] = jnp.maximum(s_max[...], pm_ref[0])
    s_cnt[...] += pc_ref[0]

  @pl.when(i == NW - 1)
  def _():
    cnt = s_cnt[:, 0:1]
    mean = s_sum[...] / jnp.maximum(cnt, 1.0)
    mx = jnp.where(cnt > 0.0, s_max[...], 0.0)
    z = jnp.concatenate([mean, mx], axis=1)
    z = jnp.maximum(jnp.dot(z, w1_ref[...], preferred_element_type=jnp.float32)
                    + b1_ref[...], 0.0)
    z = jnp.maximum(jnp.dot(z, w2_ref[...], preferred_element_type=jnp.float32)
                    + b2_ref[...], 0.0)
    out_ref[...] = (jnp.dot(z, wo_ref[...], preferred_element_type=jnp.float32)
                    + bo_ref[...])


_head = pl.pallas_call(
    _head_body,
    grid=(NW,),
    in_specs=[
        pl.BlockSpec((1, B, H), lambda i: (i, 0, 0)),
        pl.BlockSpec((1, B, H), lambda i: (i, 0, 0)),
        pl.BlockSpec((1, B, L), lambda i: (i, 0, 0)),
        pl.BlockSpec((2 * H, H), lambda i: (0, 0)),
        pl.BlockSpec((1, H), lambda i: (0, 0)),
        pl.BlockSpec((H, H // 2), lambda i: (0, 0)),
        pl.BlockSpec((1, H // 2), lambda i: (0, 0)),
        pl.BlockSpec((H // 2, T), lambda i: (0, 0)),
        pl.BlockSpec((1, T), lambda i: (0, 0)),
    ],
    out_specs=pl.BlockSpec((B, T), lambda i: (0, 0)),
    out_shape=jax.ShapeDtypeStruct((B, T), jnp.float32),
    scratch_shapes=[
        pltpu.VMEM((B, H), jnp.float32),
        pltpu.VMEM((B, H), jnp.float32),
        pltpu.VMEM((B, L), jnp.float32),
    ],
)


def kernel(x, edge_index, batch, W0, b0, g0, be0, W1, b1, g1, be1,
           W2, b2, g2, be2, fc1W, fc1b, fc2W, fc2b, foW, fob):
  src = edge_index[0]
  dst = edge_index[1]
  _deg_kernel, _scatter_kernel, _pool_kernel = _sc_kernels()

  parts = _deg_kernel(dst)
  dinv, hw = _prep(x, parts.T, W0)

  layer_params = ((b0, g0, be0, W1), (b1, g1, be1, W2), (b2, g2, be2, None))
  h3 = None
  for li, (b, g, be, Wn) in enumerate(layer_params):
    agg = _scatter_kernel(hw, dst, src)
    z, stats = _post(agg, hw, dinv, b.reshape(1, H))
    if Wn is not None:
      hw = _bn(z, stats, g.reshape(1, H), be.reshape(1, H), Wn, dinv)
    else:
      h3 = _bn_last(z, stats, g.reshape(1, H), be.reshape(1, H))

  psum, pmax, pcnt = _pool_kernel(h3, batch)
  return _head(psum, pmax, pcnt,
               fc1W, fc1b.reshape(1, H), fc2W, fc2b.reshape(1, H // 2),
               foW, fob.reshape(1, T))


# speedup vs baseline: 1.3163x; 1.1910x over previous
import functools

import jax
import jax.numpy as jnp
from jax import lax
from jax.experimental import pallas as pl
from jax.experimental.pallas import tpu as pltpu
from jax.experimental.pallas import tpu_sc as plsc

N = 50000
E = 800000
F_IN = 16
H = 128
B = 256
T = 5

NC = 2
NS = 16
L = 16
NW = NC * NS

W_T = 784
ACC_T = W_T + 8
DUMP = W_T
ROUNDS = 2
NPAD = ROUNDS * NW * W_T
G = 64
CH = 1536
NCHF = E // CH
TAIL = E - NCHF * CH
MB = CH + 2 * G
SPC = 65536

def _worker_id():
  return lax.axis_index("s") * NC + lax.axis_index("c")


EPT = E // NW
_EPT_FULL = (EPT // L) * L
_EPT_REM = EPT - _EPT_FULL


def _deg_body(dst_hbm, part_hbm, ebuf, hist):
  w = _worker_id()
  zv = jnp.zeros((L,), jnp.float32)

  def zero_body(i, _):
    hist[pl.ds(i * L, L)] = zv
    return 0

  lax.fori_loop(0, N // L, zero_body, 0, unroll=4)

  pltpu.sync_copy(dst_hbm.at[pl.ds(w * EPT, EPT)], ebuf.at[pl.ds(0, EPT)])
  ones = jnp.ones((L,), jnp.float32)

  def edge_body(i, _):
    idx = ebuf[pl.ds(i * L, L)]
    plsc.addupdate_scatter(hist, [idx], ones)
    return 0

  lax.fori_loop(0, _EPT_FULL // L, edge_body, 0, unroll=4)
  lanes = lax.iota(jnp.int32, L)
  m = lanes < _EPT_REM
  idx = jnp.where(m, ebuf[pl.ds(_EPT_FULL, L)], 0)
  plsc.addupdate_scatter(hist, [idx], ones, mask=m)

  pltpu.sync_copy(hist, part_hbm.at[w])


def _scatter_body(hw_hbm, dst_hbm, src_hbm, agg_hbm, ssrc_hbm, sdst_hbm,
                  dbuf, sbuf, mdst, msrc, m1dst, m1src, idx2, rowbuf, acc,
                  gsem0, gsem1, csem0, csem1):
  w = _worker_id()
  zv = jnp.zeros((L,), jnp.float32)
  dump_i = jnp.full((L,), DUMP, jnp.int32)
  zero_i = jnp.zeros((L,), jnp.int32)
  csems = (csem0, csem1)

  def issue_chunk(ci, b):
    off = ci * CH
    pltpu.async_copy(dst_hbm.at[pl.ds(off, CH)], dbuf.at[b], csems[b])
    pltpu.async_copy(src_hbm.at[pl.ds(off, CH)], sbuf.at[b], csems[b])

  def wait_chunk(ci, b):
    off = ci * CH
    pltpu.make_async_copy(dst_hbm.at[pl.ds(off, CH)], dbuf.at[b], csems[b]).wait()
    pltpu.make_async_copy(src_hbm.at[pl.ds(off, CH)], sbuf.at[b], csems[b]).wait()

  def accumulate(b, j):
    def acc_k8(k8, _):
      dl = mdst[pl.ds(j * G + k8 * L, L)]
      for k in range(L):
        d = dl[k]
        row = k8 * L + k
        for q in range(H // L):
          plsc.addupdate(acc.at[d, pl.ds(q * L, L)],
                         rowbuf[b, row, pl.ds(q * L, L)])
      return 0

    lax.fori_loop(0, G // L, acc_k8, 0)

  def stage_idx(j, b):
    base = j * G
    for k in range(G // L):
      idx2[b, pl.ds(k * L, L)] = msrc[pl.ds(base + k * L, L)]

  def make_flush_pairs(ng):
    def flush_pair(p, _):
      j0 = 2 * p
      j1 = j0 + 1
      stage_idx(j0, 0)
      g0 = pltpu.async_copy(hw_hbm.at[idx2.at[0]], rowbuf.at[0], gsem0)

      @pl.when(j1 < ng)
      def _():
        stage_idx(j1, 1)
        pltpu.async_copy(hw_hbm.at[idx2.at[1]], rowbuf.at[1], gsem1)

      g0.wait()
      accumulate(0, j0)

      @pl.when(j1 < ng)
      def _():
        pltpu.make_async_copy(hw_hbm.at[idx2.at[1]], rowbuf.at[1], gsem1).wait()
        accumulate(1, j1)

      return 0

    return flush_pair

  def compact(bdst, bsrc, ngf):
    @pl.when(ngf > 0)
    def _():
      base = ngf * G
      for k in range(G // L):
        td = bdst[pl.ds(base + k * L, L)]
        ts = bsrc[pl.ds(base + k * L, L)]
        bdst[pl.ds(k * L, L)] = td
        bsrc[pl.ds(k * L, L)] = ts

  def zero_acc():
    def zrow(i, _):
      for k in range(H // L):
        acc[i, pl.ds(k * L, L)] = zv
      return 0

    lax.fori_loop(0, W_T, zrow, 0, unroll=2)

  lo0 = w * W_T
  lo1 = (NW + w) * W_T

  def process_data(b, ngroups, carry):
    cur0, cur1, nsp, ovf = carry

    def filter_group(g, c):
      c0, c1 = c
      d = dbuf[b, pl.ds(g * L, L)]
      sv = sbuf[b, pl.ds(g * L, L)]
      dl0 = d - lo0
      m0 = plsc.bitcast(dl0, jnp.uint32) < jnp.uint32(W_T)
      plsc.store_compressed(mdst.at[pl.ds(c0, L)], dl0, mask=m0)
      plsc.store_compressed(msrc.at[pl.ds(c0, L)], sv, mask=m0)
      dl1 = d - lo1
      m1 = plsc.bitcast(dl1, jnp.uint32) < jnp.uint32(W_T)
      plsc.store_compressed(m1dst.at[pl.ds(c1, L)], dl1, mask=m1)
      plsc.store_compressed(m1src.at[pl.ds(c1, L)], sv, mask=m1)
      return (c0 + jnp.sum(m0.astype(jnp.int32)),
              c1 + jnp.sum(m1.astype(jnp.int32)))

    cur0, cur1 = lax.fori_loop(0, ngroups, filter_group, (cur0, cur1))
    ngf = cur0 // G
    lax.fori_loop(0, (ngf + 1) // 2, make_flush_pairs(ngf), 0)
    compact(mdst, msrc, ngf)
    cur0 = cur0 - ngf * G
    n1f = cur1 // G

    def spill_block(j, st):
      nsp_j, ovf_j = st
      ok = nsp_j + G <= SPC

      @pl.when(ok)
      def _():
        o = pl.multiple_of(nsp_j, G)
        pltpu.sync_copy(m1src.at[pl.ds(j * G, G)], ssrc_hbm.at[w, pl.ds(o, G)])
        pltpu.sync_copy(m1dst.at[pl.ds(j * G, G)], sdst_hbm.at[w, pl.ds(o, G)])

      return (jnp.where(ok, nsp_j + G, nsp_j),
              jnp.where(ok, ovf_j, jnp.int32(1)))

    nsp, ovf = lax.fori_loop(0, n1f, spill_block, (nsp, ovf))
    compact(m1dst, m1src, n1f)
    cur1 = cur1 - n1f * G
    return (cur0, cur1, nsp, ovf)

  zero_acc()
  issue_chunk(0, 0)

  def chunk_pair(p, carry):
    c0 = 2 * p
    wait_chunk(c0, 0)
    issue_chunk(c0 + 1, 1)
    carry = process_data(0, CH // L, carry)
    wait_chunk(c0 + 1, 1)

    @pl.when(c0 + 2 < NCHF)
    def _():
      issue_chunk(c0 + 2, 0)

    carry = process_data(1, CH // L, carry)
    return carry

  carry0 = (jnp.int32(0), jnp.int32(0), jnp.int32(0), jnp.int32(0))
  carry = lax.fori_loop(0, NCHF // 2, chunk_pair, carry0)
  pltpu.sync_copy(dst_hbm.at[pl.ds(NCHF * CH, TAIL)], dbuf.at[0, pl.ds(0, TAIL)])
  pltpu.sync_copy(src_hbm.at[pl.ds(NCHF * CH, TAIL)], sbuf.at[0, pl.ds(0, TAIL)])
  cur0, cur1, nsp, ovf = process_data(0, TAIL // L, carry)

  for k in range(G // L):
    mdst[pl.ds(cur0 + k * L, L)] = dump_i
    msrc[pl.ds(cur0 + k * L, L)] = zero_i
  ng_last = (cur0 + (G - 1)) // G
  lax.fori_loop(0, ng_last, make_flush_pairs(ng_last), 0)
  pltpu.sync_copy(acc.at[pl.ds(0, W_T)], agg_hbm.at[pl.ds(lo0, W_T)])

  for k in range(G // L):
    m1dst[pl.ds(cur1 + k * L, L)] = dump_i
    m1src[pl.ds(cur1 + k * L, L)] = zero_i
  ok_last = nsp + G <= SPC

  @pl.when(ok_last)
  def _():
    o = pl.multiple_of(nsp, G)
    pltpu.sync_copy(m1src.at[pl.ds(0, G)], ssrc_hbm.at[w, pl.ds(o, G)])
    pltpu.sync_copy(m1dst.at[pl.ds(0, G)], sdst_hbm.at[w, pl.ds(o, G)])

  nsp = jnp.where(ok_last, nsp + G, nsp)
  ovf = jnp.where(ok_last, ovf, jnp.int32(1))

  zero_acc()

  @pl.when(ovf == 0)
  def _consume():
    nblk = nsp // G

    def consume_pair(p, _):
      bp = pl.multiple_of(p * (2 * G), 2 * G)
      pltpu.sync_copy(ssrc_hbm.at[w, pl.ds(bp, 2 * G)], msrc.at[pl.ds(0, 2 * G)])
      pltpu.sync_copy(sdst_hbm.at[w, pl.ds(bp, 2 * G)], mdst.at[pl.ds(0, 2 * G)])
      ng_here = jnp.minimum(nblk - 2 * p, 2)
      lax.fori_loop(0, 1, make_flush_pairs(ng_here), 0)
      return 0

    lax.fori_loop(0, (nblk + 1) // 2, consume_pair, 0)

  @pl.when(ovf != 0)
  def _fallback():
    def fb_chunk(ci, cur):
      clen = jnp.minimum(E - ci * CH, CH)
      del clen
      pltpu.sync_copy(dst_hbm.at[pl.ds(ci * CH, CH)], dbuf.at[0])
      pltpu.sync_copy(src_hbm.at[pl.ds(ci * CH, CH)], sbuf.at[0])

      def fb_filter(g, c):
        d = dbuf[0, pl.ds(g * L, L)]
        sv = sbuf[0, pl.ds(g * L, L)]
        dl1 = d - lo1
        m1 = plsc.bitcast(dl1, jnp.uint32) < jnp.uint32(W_T)
        plsc.store_compressed(mdst.at[pl.ds(c, L)], dl1, mask=m1)
        plsc.store_compressed(msrc.at[pl.ds(c, L)], sv, mask=m1)
        return c + jnp.sum(m1.astype(jnp.int32))

      cur = lax.fori_loop(0, CH // L, fb_filter, cur)
      ngf = cur // G
      lax.fori_loop(0, (ngf + 1) // 2, make_flush_pairs(ngf), 0)
      compact(mdst, msrc, ngf)
      return cur - ngf * G

    cur = lax.fori_loop(0, NCHF, fb_chunk, jnp.int32(0))
    pltpu.sync_copy(dst_hbm.at[pl.ds(NCHF * CH, TAIL)], dbuf.at[0, pl.ds(0, TAIL)])
    pltpu.sync_copy(src_hbm.at[pl.ds(NCHF * CH, TAIL)], sbuf.at[0, pl.ds(0, TAIL)])

    def fb_filter2(g, c):
      d = dbuf[0, pl.ds(g * L, L)]
      sv = sbuf[0, pl.ds(g * L, L)]
      dl1 = d - lo1
      m1 = plsc.bitcast(dl1, jnp.uint32) < jnp.uint32(W_T)
      plsc.store_compressed(mdst.at[pl.ds(c, L)], dl1, mask=m1)
      plsc.store_compressed(msrc.at[pl.ds(c, L)], sv, mask=m1)
      return c + jnp.sum(m1.astype(jnp.int32))

    cur = lax.fori_loop(0, TAIL // L, fb_filter2, cur)
    for k in range(G // L):
      mdst[pl.ds(cur + k * L, L)] = dump_i
      msrc[pl.ds(cur + k * L, L)] = zero_i
    ngl = (cur + (G - 1)) // G
    lax.fori_loop(0, ngl, make_flush_pairs(ngl), 0)

  pltpu.sync_copy(acc.at[pl.ds(0, W_T)], agg_hbm.at[pl.ds(lo1, W_T)])


NPT = 1552
PTAIL = N - NPT * NW
PROWS = B + 8
CHP = 128


def _pool_body(h_hbm, batch_hbm, psum_hbm, pmax_hbm, pcnt_hbm,
               hbuf, bbv, asum, amax, acnt):
  w = _worker_id()
  zv = jnp.zeros((L,), jnp.float32)
  ninf = jnp.full((L,), -3.0e38, jnp.float32)
  onev = jnp.ones((L,), jnp.float32)

  def init_body(i, _):
    for k in range(H // L):
      asum[i, pl.ds(k * L, L)] = zv
      amax[i, pl.ds(k * L, L)] = ninf
    acnt[i, pl.ds(0, L)] = zv
    return 0

  lax.fori_loop(0, PROWS, init_body, 0, unroll=2)

  def do_chunk(base, nrows):
    pltpu.sync_copy(h_hbm.at[pl.ds(base, nrows)], hbuf.at[pl.ds(0, nrows)])
    pltpu.sync_copy(batch_hbm.at[pl.ds(base, nrows)], bbv.at[pl.ds(0, nrows)])

    def group_body(g, _):
      bvec = bbv[pl.ds(g * L, L)]
      for k in range(L):
        b = bvec[k]
        row = g * L + k
        for q in range(H // L):
          v = hbuf[row, pl.ds(q * L, L)]
          asum[b, pl.ds(q * L, L)] = asum[b, pl.ds(q * L, L)] + v
          amax[b, pl.ds(q * L, L)] = jnp.maximum(amax[b, pl.ds(q * L, L)], v)
        acnt[b, pl.ds(0, L)] = acnt[b, pl.ds(0, L)] + onev
      return 0

    lax.fori_loop(0, nrows // L, group_body, 0)

  base0 = w * NPT

  def chunk_body(ci, _):
    do_chunk(base0 + ci * CHP, CHP)
    return 0

  lax.fori_loop(0, NPT // CHP, chunk_body, 0)
  do_chunk(base0 + (NPT // CHP) * CHP, NPT - (NPT // CHP) * CHP)

  @pl.when(w == NW - 1)
  def _tail():
    tb = NW * NPT

    def tail_chunk(ci, _):
      do_chunk(tb + ci * CHP, CHP)
      return 0

    lax.fori_loop(0, PTAIL // CHP, tail_chunk, 0)
    do_chunk(tb + (PTAIL // CHP) * CHP, PTAIL - (PTAIL // CHP) * CHP)

  pltpu.sync_copy(asum, psum_hbm.at[w])
  pltpu.sync_copy(amax, pmax_hbm.at[w])
  pltpu.sync_copy(acnt, pcnt_hbm.at[w])


@functools.cache
def _sc_kernels():
  mesh = plsc.VectorSubcoreMesh(
      core_axis_name="c", subcore_axis_name="s",
      num_cores=NC, num_subcores=NS)
  cp = pltpu.CompilerParams(
      needs_layout_passes=False, use_tc_tiling_on_sc=False)
  deg = pl.kernel(
      _deg_body,
      out_type=jax.ShapeDtypeStruct((NW, N), jnp.float32),
      mesh=mesh,
      scratch_types=[
          pltpu.VMEM((_EPT_FULL + L,), jnp.int32),
          pltpu.VMEM((N,), jnp.float32),
      ],
      compiler_params=cp,
  )
  scatter = pl.kernel(
      _scatter_body,
      out_type=(
          jax.ShapeDtypeStruct((NPAD, H), jnp.float32),
          jax.ShapeDtypeStruct((NW, SPC), jnp.int32),
          jax.ShapeDtypeStruct((NW, SPC), jnp.int32),
      ),
      mesh=mesh,
      scratch_types=[
          pltpu.VMEM((2, CH), jnp.int32),
          pltpu.VMEM((2, CH), jnp.int32),
          pltpu.VMEM((MB,), jnp.int32),
          pltpu.VMEM((MB,), jnp.int32),
          pltpu.VMEM((MB,), jnp.int32),
          pltpu.VMEM((MB,), jnp.int32),
          pltpu.VMEM((2, G), jnp.int32),
          pltpu.VMEM((2, G, H), jnp.float32),
          pltpu.VMEM((ACC_T, H), jnp.float32),
          pltpu.SemaphoreType.DMA,
          pltpu.SemaphoreType.DMA,
          pltpu.SemaphoreType.DMA,
          pltpu.SemaphoreType.DMA,
      ],
      compiler_params=cp,
  )
  pool = pl.kernel(
      _pool_body,
      out_type=(
          jax.ShapeDtypeStruct((NW, PROWS, H), jnp.float32),
          jax.ShapeDtypeStruct((NW, PROWS, H), jnp.float32),
          jax.ShapeDtypeStruct((NW, PROWS, L), jnp.float32),
      ),
      mesh=mesh,
      scratch_types=[
          pltpu.VMEM((CHP, H), jnp.float32),
          pltpu.VMEM((CHP,), jnp.int32),
          pltpu.VMEM((PROWS, H), jnp.float32),
          pltpu.VMEM((PROWS, H), jnp.float32),
          pltpu.VMEM((PROWS, L), jnp.float32),
      ],
      compiler_params=cp,
  )
  return deg, scatter, pool


RB = 1000
GRID = N // RB


def _prep_body(x_ref, pt_ref, w0_ref, dinv_ref, hw_ref):
  deg = jnp.sum(pt_ref[...], axis=1, keepdims=True) + 1.0
  dinv = lax.rsqrt(jnp.maximum(deg, 1.0))
  dinv_ref[...] = dinv
  hw = jnp.dot(x_ref[...], w0_ref[...], preferred_element_type=jnp.float32)
  hw_ref[...] = hw * dinv


_prep = pl.pallas_call(
    _prep_body,
    grid=(GRID,),
    in_specs=[
        pl.BlockSpec((RB, F_IN), lambda i: (i, 0)),
        pl.BlockSpec((RB, NW), lambda i: (i, 0)),
        pl.BlockSpec((F_IN, H), lambda i: (0, 0)),
    ],
    out_specs=[
        pl.BlockSpec((RB, 1), lambda i: (i, 0)),
        pl.BlockSpec((RB, H), lambda i: (i, 0)),
    ],
    out_shape=[
        jax.ShapeDtypeStruct((N, 1), jnp.float32),
        jax.ShapeDtypeStruct((N, H), jnp.float32),
    ],
)


def _post_body(agg_ref, hwp_ref, dinv_ref, b_ref, z_ref, st_ref):
  i = pl.program_id(0)
  zv = dinv_ref[...] * (agg_ref[...] + hwp_ref[...]) + b_ref[...]
  z_ref[...] = zv

  @pl.when(i == 0)
  def _():
    st_ref[...] = jnp.zeros((8, H), jnp.float32)

  st_ref[0:1, :] += jnp.sum(zv, axis=0, keepdims=True)
  st_ref[1:2, :] += jnp.sum(zv * zv, axis=0, keepdims=True)


_post = pl.pallas_call(
    _post_body,
    grid=(GRID,),
    in_specs=[
        pl.BlockSpec((RB, H), lambda i: (i, 0)),
        pl.BlockSpec((RB, H), lambda i: (i, 0)),
        pl.BlockSpec((RB, 1), lambda i: (i, 0)),
        pl.BlockSpec((1, H), lambda i: (0, 0)),
    ],
    out_specs=[
        pl.BlockSpec((RB, H), lambda i: (i, 0)),
        pl.BlockSpec((8, H), lambda i: (0, 0)),
    ],
    out_shape=[
        jax.ShapeDtypeStruct((N, H), jnp.float32),
        jax.ShapeDtypeStruct((8, H), jnp.float32),
    ],
)


def _bn_common(z_ref, st_ref, g_ref, be_ref):
  mu = st_ref[0:1, :] * (1.0 / N)
  ms = st_ref[1:2, :] * (1.0 / N)
  var = ms - mu * mu
  scale = g_ref[...] * lax.rsqrt(var + 1e-5)
  return jnp.maximum((z_ref[...] - mu) * scale + be_ref[...], 0.0)


def _bn_body(z_ref, st_ref, g_ref, be_ref, w_ref, dinv_ref, hwn_ref):
  h = _bn_common(z_ref, st_ref, g_ref, be_ref)
  hwn_ref[...] = jnp.dot(h, w_ref[...], preferred_element_type=jnp.float32) * dinv_ref[...]


_bn = pl.pallas_call(
    _bn_body,
    grid=(GRID,),
    in_specs=[
        pl.BlockSpec((RB, H), lambda i: (i, 0)),
        pl.BlockSpec((8, H), lambda i: (0, 0)),
        pl.BlockSpec((1, H), lambda i: (0, 0)),
        pl.BlockSpec((1, H), lambda i: (0, 0)),
        pl.BlockSpec((H, H), lambda i: (0, 0)),
        pl.BlockSpec((RB, 1), lambda i: (i, 0)),
    ],
    out_specs=pl.BlockSpec((RB, H), lambda i: (i, 0)),
    out_shape=jax.ShapeDtypeStruct((N, H), jnp.float32),
)


def _bn_last_body(z_ref, st_ref, g_ref, be_ref, h_ref):
  h_ref[...] = _bn_common(z_ref, st_ref, g_ref, be_ref)


_bn_last = pl.pallas_call(
    _bn_last_body,
    grid=(GRID,),
    in_specs=[
        pl.BlockSpec((RB, H), lambda i: (i, 0)),
        pl.BlockSpec((8, H), lambda i: (0, 0)),
        pl.BlockSpec((1, H), lambda i: (0, 0)),
        pl.BlockSpec((1, H), lambda i: (0, 0)),
    ],
    out_specs=pl.BlockSpec((RB, H), lambda i: (i, 0)),
    out_shape=jax.ShapeDtypeStruct((N, H), jnp.float32),
)


def _head_body(ps_ref, pm_ref, pc_ref, w1_ref, b1_ref, w2_ref, b2_ref,
               wo_ref, bo_ref, out_ref, s_sum, s_max, s_cnt):
  i = pl.program_id(0)

  @pl.when(i == 0)
  def _():
    s_sum[...] = ps_ref[0]
    s_max[...] = pm_ref[0]
    s_cnt[...] = pc_ref[0]

  @pl.when(i > 0)
  def _():
    s_sum[...] += ps_ref[0]
    s_max[...] = jnp.maximum(s_max[...], pm_ref[0])
    s_cnt[...] += pc_ref[0]

  @pl.when(i == NW - 1)
  def _():
    cnt = s_cnt[:, 0:1]
    mean = s_sum[...] / jnp.maximum(cnt, 1.0)
    mx = jnp.where(cnt > 0.0, s_max[...], 0.0)
    z = jnp.concatenate([mean, mx], axis=1)
    z = jnp.maximum(jnp.dot(z, w1_ref[...], preferred_element_type=jnp.float32)
                    + b1_ref[...], 0.0)
    z = jnp.maximum(jnp.dot(z, w2_ref[...], preferred_element_type=jnp.float32)
                    + b2_ref[...], 0.0)
    out_ref[...] = (jnp.dot(z, wo_ref[...], preferred_element_type=jnp.float32)
                    + bo_ref[...])


_head = pl.pallas_call(
    _head_body,
    grid=(NW,),
    in_specs=[
        pl.BlockSpec((1, B, H), lambda i: (i, 0, 0)),
        pl.BlockSpec((1, B, H), lambda i: (i, 0, 0)),
        pl.BlockSpec((1, B, L), lambda i: (i, 0, 0)),
        pl.BlockSpec((2 * H, H), lambda i: (0, 0)),
        pl.BlockSpec((1, H), lambda i: (0, 0)),
        pl.BlockSpec((H, H // 2), lambda i: (0, 0)),
        pl.BlockSpec((1, H // 2), lambda i: (0, 0)),
        pl.BlockSpec((H // 2, T), lambda i: (0, 0)),
        pl.BlockSpec((1, T), lambda i: (0, 0)),
    ],
    out_specs=pl.BlockSpec((B, T), lambda i: (0, 0)),
    out_shape=jax.ShapeDtypeStruct((B, T), jnp.float32),
    scratch_shapes=[
        pltpu.VMEM((B, H), jnp.float32),
        pltpu.VMEM((B, H), jnp.float32),
        pltpu.VMEM((B, L), jnp.float32),
    ],
)


def kernel(x, edge_index, batch, W0, b0, g0, be0, W1, b1, g1, be1,
           W2, b2, g2, be2, fc1W, fc1b, fc2W, fc2b, foW, fob):
  src = edge_index[0]
  dst = edge_index[1]
  _deg_kernel, _scatter_kernel, _pool_kernel = _sc_kernels()

  parts = _deg_kernel(dst)
  dinv, hw = _prep(x, parts.T, W0)

  layer_params = ((b0, g0, be0, W1), (b1, g1, be1, W2), (b2, g2, be2, None))
  h3 = None
  for li, (b, g, be, Wn) in enumerate(layer_params):
    agg, _, _ = _scatter_kernel(hw, dst, src)
    z, stats = _post(agg, hw, dinv, b.reshape(1, H))
    if Wn is not None:
      hw = _bn(z, stats, g.reshape(1, H), be.reshape(1, H), Wn, dinv)
    else:
      h3 = _bn_last(z, stats, g.reshape(1, H), be.reshape(1, H))

  psum, pmax, pcnt = _pool_kernel(h3, batch)
  return _head(psum, pmax, pcnt,
               fc1W, fc1b.reshape(1, H), fc2W, fc2b.reshape(1, H // 2),
               foW, fob.reshape(1, T))
